# baseline (speedup 1.0000x reference)
import dataclasses
import functools

import jax
import jax.numpy as jnp
from jax import lax
from jax.experimental import pallas as pl
from jax.experimental.pallas import tpu as pltpu
from jax.experimental.pallas import tpu_sc as plsc

N = 10000
NP = 10240
E = 320000
D_IN = 128
NSC = 2
NTEC = 16
NW = NSC * NTEC
EPW = E // NW
STRIPE = NP // NTEC
KT = 16

f32 = jnp.float32


def _elu(x):
    return jnp.where(x > 0, x, jnp.exp(x) - 1.0)


def _att_mat(att):
    H, C = att.shape
    m = jnp.zeros((H * C, 16), f32)
    for h in range(H):
        m = m.at[h * C:(h + 1) * C, h].set(att[h])
    return m


def _prep_body(H, HC, x_ref, w_ref, as_m_ref, ad_m_ref,
               o_t, o_ad, o_m2, o_init):
    x = x_ref[...]
    h = jnp.dot(x, w_ref[...], preferred_element_type=f32)
    a_s = jnp.dot(h, as_m_ref[...], preferred_element_type=f32)
    a_d = jnp.dot(h, ad_m_ref[...], preferred_element_type=f32)
    o_t[...] = jnp.concatenate([h, a_s], axis=1)
    o_ad[...] = a_d
    lane = lax.broadcasted_iota(jnp.int32, (1, 16), 1)
    m = jnp.max(a_s, axis=0, keepdims=True) + jnp.max(a_d, axis=0, keepdims=True)
    m2 = jnp.where(lane < H, jnp.maximum(m, 0.0), 1e30)
    o_m2[...] = m2
    e0 = a_s + a_d
    p0 = jnp.exp(jnp.maximum(e0, 0.2 * e0) - m2)
    C = HC // H
    parts = [h[:, j * C:(j + 1) * C] * p0[:, j:j + 1] for j in range(H)]
    io = jnp.concatenate(parts + [p0], axis=1)
    o_init[...] = jnp.concatenate([io, jnp.zeros_like(io)], axis=0)


_TC_PARAMS = pltpu.CompilerParams(vmem_limit_bytes=100 * 1024 * 1024)


def _make_prep(H, HC, F):
    return pl.pallas_call(
        functools.partial(_prep_body, H, HC),
        compiler_params=_TC_PARAMS,
        out_shape=(
            jax.ShapeDtypeStruct((NP, HC + 16), f32),
            jax.ShapeDtypeStruct((NP, 16), f32),
            jax.ShapeDtypeStruct((1, 16), f32),
            jax.ShapeDtypeStruct((2 * NP, HC + 16), f32),
        ),
    )


def _combine_body(H, HC, apply_elu, acc_ref, b_ref, o_ref):
    full = acc_ref[0:NP, :] + acc_ref[NP:2 * NP, :]
    a = full[:, 0:HC]
    d = full[:, HC:HC + 16]
    C = HC // H
    parts = [a[:, j * C:(j + 1) * C] / (d[:, j:j + 1] + 1e-16) for j in range(H)]
    o = jnp.concatenate(parts, axis=1) + b_ref[...]
    if apply_elu:
        o = _elu(o)
    o_ref[...] = o


def _make_combine(H, HC, apply_elu):
    return pl.pallas_call(
        functools.partial(_combine_body, H, HC, apply_elu),
        compiler_params=_TC_PARAMS,
        out_shape=jax.ShapeDtypeStruct((NP, HC), f32),
    )


_GDN = lax.GatherDimensionNumbers(offset_dims=(), collapsed_slice_dims=(0,),
                                  start_index_map=(0,))


def _lane_bcast(p, idx):
    return lax.gather(p, idx, _GDN, (1,),
                      mode=lax.GatherScatterMode.PROMISE_IN_BOUNDS)


def _make_sc(H, HC, KW):
    CH = HC // 16
    TW = HC + 16
    NWIN = (EPW - KT) // KW
    assert NWIN * KW + KT == EPW
    mesh = plsc.VectorSubcoreMesh(core_axis_name="c", subcore_axis_name="s")

    def body(t_hbm, ad_hbm, m2_hbm, src_hbm, dst_hbm, init_hbm,
             acc_hbm,
             acc_sp,
             src_v0, dst_v0, dsc_v0, rows_v0, ad_v0,
             src_v1, dst_v1, dsc_v1, rows_v1, ad_v1,
             src_t, dst_t, rows_t, ad_t,
             m2_v,
             si1_0, si2_0, sgr_0, sga_0, ssr_0,
             si1_1, si2_1, sgr_1, sga_1, ssr_1):
        src_v = (src_v0, src_v1)
        dst_v = (dst_v0, dst_v1)
        dsc_v = (dsc_v0, dsc_v1)
        rows_v = (rows_v0, rows_v1)
        ad_v = (ad_v0, ad_v1)
        si1 = (si1_0, si1_1)
        si2 = (si2_0, si2_1)
        sgr = (sgr_0, sgr_1)
        sga = (sga_0, sga_1)
        ssr = (ssr_0, ssr_1)

        c = lax.axis_index("c")
        s = lax.axis_index("s")
        nbase = s * STRIPE
        pltpu.sync_copy(init_hbm.at[pl.ds(c * NP + nbase, STRIPE)],
                        acc_sp.at[pl.ds(nbase, STRIPE)])
        pltpu.sync_copy(m2_hbm, m2_v)
        plsc.subcore_barrier()
        m2 = m2_v[...]
        ebase = (c * NTEC + s) * EPW
        cidx = [jnp.full((16, 1), j * H // CH, jnp.int32) for j in range(CH)]

        def issue_idx(ww, b):
            base = ebase + jnp.minimum(ww, NWIN - 1) * KW
            pltpu.async_copy(src_hbm.at[pl.ds(base, KW)], src_v[b], si1[b])
            pltpu.async_copy(dst_hbm.at[pl.ds(base, KW)], dst_v[b], si2[b])

        def wait_idx(b):
            pltpu.make_async_copy(src_hbm.at[pl.ds(0, KW)], src_v[b], si1[b]).wait()
            pltpu.make_async_copy(dst_hbm.at[pl.ds(0, KW)], dst_v[b], si2[b]).wait()

        def issue_gathers(b):
            pltpu.async_copy(t_hbm.at[src_v[b]], rows_v[b], sgr[b])
            pltpu.async_copy(ad_hbm.at[dst_v[b]], ad_v[b], sga[b])

        def wait_gathers(b):
            pltpu.make_async_copy(t_hbm.at[src_v[b]], rows_v[b], sgr[b]).wait()
            pltpu.make_async_copy(ad_hbm.at[dst_v[b]], ad_v[b], sga[b]).wait()

        def copy_dst(b):
            for i in range(KW // 16):
                dsc_v[b][pl.ds(i * 16, 16)] = dst_v[b][pl.ds(i * 16, 16)]

        def compute(b):
            rows_r, ad_r = rows_v[b], ad_v[b]

            @pl.loop(0, KW)
            def _(k):
                e = rows_r[k, pl.ds(HC, 16)] + ad_r[k]
                vals = [rows_r[k, pl.ds(j * 16, 16)] for j in range(CH)]
                p = jnp.exp(jnp.maximum(e, 0.2 * e) - m2)
                rows_r[k, pl.ds(HC, 16)] = p
                prods = [vals[j] * _lane_bcast(p, cidx[j]) for j in range(CH)]
                for j in range(CH):
                    rows_r[k, pl.ds(j * 16, 16)] = prods[j]

        def issue_scat(b):
            pltpu.async_copy(rows_v[b], acc_sp.at[dsc_v[b]], ssr[b], add=True)

        def wait_scat(b):
            pltpu.make_async_copy(rows_v[b], acc_sp.at[dsc_v[b]], ssr[b]).wait()

        def stage(ww, b, first):
            if not first:
                wait_scat(b)
            wait_idx(b)
            issue_gathers(b)
            issue_idx(ww + 1, b ^ 1)
            wait_gathers(b)
            copy_dst(b)
            compute(b)
            issue_scat(b)

        issue_idx(0, 0)
        stage(0, 0, True)
        stage(1, 1, True)

        @pl.loop(2, NWIN, step=2)
        def _(w):
            stage(w, 0, False)
            stage(w + 1, 1, False)

        wait_scat(0)
        wait_scat(1)
        wait_idx(0)

        tbase = ebase + NWIN * KW
        pltpu.sync_copy(src_hbm.at[pl.ds(tbase, KT)], src_t)
        pltpu.sync_copy(dst_hbm.at[pl.ds(tbase, KT)], dst_t)
        pltpu.sync_copy(t_hbm.at[src_t], rows_t)
        pltpu.sync_copy(ad_hbm.at[dst_t], ad_t)

        @pl.loop(0, KT)
        def _(k):
            e = rows_t[k, pl.ds(HC, 16)] + ad_t[k]
            vals = [rows_t[k, pl.ds(j * 16, 16)] for j in range(CH)]
            p = jnp.exp(jnp.maximum(e, 0.2 * e) - m2)
            rows_t[k, pl.ds(HC, 16)] = p
            prods = [vals[j] * _lane_bcast(p, cidx[j]) for j in range(CH)]
            for j in range(CH):
                rows_t[k, pl.ds(j * 16, 16)] = prods[j]

        pltpu.sync_copy(rows_t, acc_sp.at[dst_t], add=True)

        plsc.subcore_barrier()
        pltpu.sync_copy(acc_sp.at[pl.ds(nbase, STRIPE)],
                        acc_hbm.at[pl.ds(c * NP + nbase, STRIPE)])

    cp = pltpu.CompilerParams(needs_layout_passes=False,
                              use_tc_tiling_on_sc=False)
    return pl.kernel(
        body,
        out_type=jax.ShapeDtypeStruct((2 * NP, TW), f32),
        mesh=mesh,
        compiler_params=cp,
        scratch_types=(
            [pltpu.VMEM_SHARED((NP, TW), f32)]
            + 2 * [pltpu.VMEM((KW,), jnp.int32), pltpu.VMEM((KW,), jnp.int32),
                   pltpu.VMEM((KW,), jnp.int32),
                   pltpu.VMEM((KW, TW), f32),
                   pltpu.VMEM((KW, 16), f32)]
            + [pltpu.VMEM((KT,), jnp.int32), pltpu.VMEM((KT,), jnp.int32),
               pltpu.VMEM((KT, TW), f32),
               pltpu.VMEM((KT, 16), f32),
               pltpu.VMEM((16,), f32)]
            + 10 * [pltpu.SemaphoreType.DMA]
        ),
    )


def _gat_layer(prep, sc, combine, x_p, W, A_s, A_d, bias, src, dst):
    t, a_d, m2, init = prep(x_p, W, A_s, A_d)
    acc = sc(t, a_d, m2.reshape(16), src, dst, init)
    return combine(acc, bias)


_prep1 = _make_prep(8, 128, 128)
_prep2 = _make_prep(1, 64, 128)
_sc1 = _make_sc(8, 128, 64)
_sc2 = _make_sc(1, 64, 64)
_comb1 = _make_combine(8, 128, True)
_comb2 = _make_combine(1, 64, False)


def kernel(x, edge_index, W1, att_src1, att_dst1, bias1,
           W2, att_src2, att_dst2, bias2):
    src = edge_index[0].astype(jnp.int32)
    dst = edge_index[1].astype(jnp.int32)
    x_p = jnp.zeros((NP, D_IN), f32).at[:N].set(x)
    out1 = _gat_layer(_prep1, _sc1, _comb1, x_p, W1,
                      _att_mat(att_src1), _att_mat(att_dst1),
                      bias1.reshape(1, 128), src, dst)
    out2 = _gat_layer(_prep2, _sc2, _comb2, out1, W2,
                      _att_mat(att_src2), _att_mat(att_dst2),
                      bias2.reshape(1, 64), src, dst)
    return out2[:N]


# speedup vs baseline: 1.0887x; 1.0887x over previous
import dataclasses
import functools

import jax
import jax.numpy as jnp
from jax import lax
from jax.experimental import pallas as pl
from jax.experimental.pallas import tpu as pltpu
from jax.experimental.pallas import tpu_sc as plsc

N = 10000
NP = 10240
E = 320000
D_IN = 128
NSC = 2
NTEC = 16
NW = NSC * NTEC
EPW = E // NW
STRIPE = NP // NTEC
KT = 16

f32 = jnp.float32


def _elu(x):
    return jnp.where(x > 0, x, jnp.exp(x) - 1.0)


def _att_mat(att):
    H, C = att.shape
    m = jnp.zeros((H * C, 16), f32)
    for h in range(H):
        m = m.at[h * C:(h + 1) * C, h].set(att[h])
    return m


def _prep_body(H, HC, x_ref, w_ref, as_m_ref, ad_m_ref,
               o_t, o_ad, o_m2, o_init):
    x = x_ref[...]
    h = jnp.dot(x, w_ref[...], preferred_element_type=f32)
    a_s = jnp.dot(h, as_m_ref[...], preferred_element_type=f32)
    a_d = jnp.dot(h, ad_m_ref[...], preferred_element_type=f32)
    o_t[...] = jnp.concatenate([h, a_s], axis=1)
    o_ad[...] = a_d
    lane = lax.broadcasted_iota(jnp.int32, (1, 16), 1)
    m = jnp.max(a_s, axis=0, keepdims=True) + jnp.max(a_d, axis=0, keepdims=True)
    m2 = jnp.where(lane < H, jnp.maximum(m, 0.0), 1e30)
    o_m2[...] = m2
    e0 = a_s + a_d
    p0 = jnp.exp(jnp.maximum(e0, 0.2 * e0) - m2)
    C = HC // H
    parts = [h[:, j * C:(j + 1) * C] * p0[:, j:j + 1] for j in range(H)]
    io = jnp.concatenate(parts + [p0], axis=1)
    o_init[...] = jnp.concatenate([io, jnp.zeros_like(io)], axis=0)


_TC_PARAMS = pltpu.CompilerParams(vmem_limit_bytes=100 * 1024 * 1024)


def _make_prep(H, HC, F):
    return pl.pallas_call(
        functools.partial(_prep_body, H, HC),
        compiler_params=_TC_PARAMS,
        out_shape=(
            jax.ShapeDtypeStruct((NP, HC + 16), f32),
            jax.ShapeDtypeStruct((NP, 16), f32),
            jax.ShapeDtypeStruct((1, 16), f32),
            jax.ShapeDtypeStruct((2 * NP, HC + 16), f32),
        ),
    )


def _combine_body(H, HC, apply_elu, acc_ref, b_ref, o_ref):
    full = acc_ref[0:NP, :] + acc_ref[NP:2 * NP, :]
    a = full[:, 0:HC]
    d = full[:, HC:HC + 16]
    C = HC // H
    parts = [a[:, j * C:(j + 1) * C] / (d[:, j:j + 1] + 1e-16) for j in range(H)]
    o = jnp.concatenate(parts, axis=1) + b_ref[...]
    if apply_elu:
        o = _elu(o)
    o_ref[...] = o


def _make_combine(H, HC, apply_elu):
    return pl.pallas_call(
        functools.partial(_combine_body, H, HC, apply_elu),
        compiler_params=_TC_PARAMS,
        out_shape=jax.ShapeDtypeStruct((NP, HC), f32),
    )


_GDN = lax.GatherDimensionNumbers(offset_dims=(), collapsed_slice_dims=(0,),
                                  start_index_map=(0,))


def _lane_bcast(p, idx):
    return lax.gather(p, idx, _GDN, (1,),
                      mode=lax.GatherScatterMode.PROMISE_IN_BOUNDS)


def _make_sc(H, HC, KW):
    CH = HC // 16
    TW = HC + 16
    NWIN = (EPW - KT) // KW
    assert NWIN * KW + KT == EPW
    mesh = plsc.VectorSubcoreMesh(core_axis_name="c", subcore_axis_name="s")

    def body(t_hbm, ad_hbm, m2_hbm, src_hbm, dst_hbm, init_hbm,
             acc_hbm,
             acc_sp,
             src_v0, dst_v0, dsc_v0, rows_v0, ad_v0,
             src_v1, dst_v1, dsc_v1, rows_v1, ad_v1,
             src_t, dst_t, rows_t, ad_t,
             m2_v,
             si1_0, si2_0, sgr_0, sga_0, ssr_0,
             si1_1, si2_1, sgr_1, sga_1, ssr_1):
        src_v = (src_v0, src_v1)
        dst_v = (dst_v0, dst_v1)
        dsc_v = (dsc_v0, dsc_v1)
        rows_v = (rows_v0, rows_v1)
        ad_v = (ad_v0, ad_v1)
        si1 = (si1_0, si1_1)
        si2 = (si2_0, si2_1)
        sgr = (sgr_0, sgr_1)
        sga = (sga_0, sga_1)
        ssr = (ssr_0, ssr_1)

        c = lax.axis_index("c")
        s = lax.axis_index("s")
        nbase = s * STRIPE
        pltpu.sync_copy(init_hbm.at[pl.ds(c * NP + nbase, STRIPE)],
                        acc_sp.at[pl.ds(nbase, STRIPE)])
        pltpu.sync_copy(m2_hbm, m2_v)
        plsc.subcore_barrier()
        m2 = m2_v[...]
        ebase = (c * NTEC + s) * EPW
        cidx = [jnp.full((16, 1), j * H // CH, jnp.int32) for j in range(CH)]

        def issue_idx(ww, b):
            base = ebase + jnp.minimum(ww, NWIN - 1) * KW
            pltpu.async_copy(src_hbm.at[pl.ds(base, KW)], src_v[b], si1[b])
            pltpu.async_copy(dst_hbm.at[pl.ds(base, KW)], dst_v[b], si2[b])

        def wait_idx(b):
            pltpu.make_async_copy(src_hbm.at[pl.ds(0, KW)], src_v[b], si1[b]).wait()
            pltpu.make_async_copy(dst_hbm.at[pl.ds(0, KW)], dst_v[b], si2[b]).wait()

        def issue_gathers(b):
            pltpu.async_copy(t_hbm.at[src_v[b]], rows_v[b], sgr[b])
            pltpu.async_copy(ad_hbm.at[dst_v[b]], ad_v[b], sga[b])

        def wait_gathers(b):
            pltpu.make_async_copy(t_hbm.at[src_v[b]], rows_v[b], sgr[b]).wait()
            pltpu.make_async_copy(ad_hbm.at[dst_v[b]], ad_v[b], sga[b]).wait()

        def copy_dst(b):
            for i in range(KW // 16):
                dsc_v[b][pl.ds(i * 16, 16)] = dst_v[b][pl.ds(i * 16, 16)]

        def compute(b):
            rows_r, ad_r = rows_v[b], ad_v[b]

            @pl.loop(0, KW)
            def _(k):
                e = rows_r[k, pl.ds(HC, 16)] + ad_r[k]
                vals = [rows_r[k, pl.ds(j * 16, 16)] for j in range(CH)]
                p = jnp.exp(jnp.maximum(e, 0.2 * e) - m2)
                rows_r[k, pl.ds(HC, 16)] = p
                prods = [vals[j] * _lane_bcast(p, cidx[j]) for j in range(CH)]
                for j in range(CH):
                    rows_r[k, pl.ds(j * 16, 16)] = prods[j]

        def issue_scat(b):
            pltpu.async_copy(rows_v[b], acc_sp.at[dsc_v[b]], ssr[b], add=True)

        def wait_scat(b):
            pltpu.make_async_copy(rows_v[b], acc_sp.at[dsc_v[b]], ssr[b]).wait()

        def stage(ww, b, first):
            if not first:
                wait_scat(b)
            wait_idx(b)
            issue_gathers(b)
            issue_idx(ww + 1, b ^ 1)
            wait_gathers(b)
            copy_dst(b)
            compute(b)
            issue_scat(b)

        issue_idx(0, 0)
        stage(0, 0, True)
        stage(1, 1, True)

        @pl.loop(2, NWIN, step=2)
        def _(w):
            stage(w, 0, False)
            stage(w + 1, 1, False)

        wait_scat(0)
        wait_scat(1)
        wait_idx(0)

        tbase = ebase + NWIN * KW
        pltpu.sync_copy(src_hbm.at[pl.ds(tbase, KT)], src_t)
        pltpu.sync_copy(dst_hbm.at[pl.ds(tbase, KT)], dst_t)
        pltpu.sync_copy(t_hbm.at[src_t], rows_t)
        pltpu.sync_copy(ad_hbm.at[dst_t], ad_t)

        @pl.loop(0, KT)
        def _(k):
            e = rows_t[k, pl.ds(HC, 16)] + ad_t[k]
            vals = [rows_t[k, pl.ds(j * 16, 16)] for j in range(CH)]
            p = jnp.exp(jnp.maximum(e, 0.2 * e) - m2)
            rows_t[k, pl.ds(HC, 16)] = p
            prods = [vals[j] * _lane_bcast(p, cidx[j]) for j in range(CH)]
            for j in range(CH):
                rows_t[k, pl.ds(j * 16, 16)] = prods[j]

        pltpu.sync_copy(rows_t, acc_sp.at[dst_t], add=True)

        plsc.subcore_barrier()
        pltpu.sync_copy(acc_sp.at[pl.ds(nbase, STRIPE)],
                        acc_hbm.at[pl.ds(c * NP + nbase, STRIPE)])

    cp = pltpu.CompilerParams(needs_layout_passes=False,
                              use_tc_tiling_on_sc=False)
    return pl.kernel(
        body,
        out_type=jax.ShapeDtypeStruct((2 * NP, TW), f32),
        mesh=mesh,
        compiler_params=cp,
        scratch_types=(
            [pltpu.VMEM_SHARED((NP, TW), f32)]
            + 2 * [pltpu.VMEM((KW,), jnp.int32), pltpu.VMEM((KW,), jnp.int32),
                   pltpu.VMEM((KW,), jnp.int32),
                   pltpu.VMEM((KW, TW), f32),
                   pltpu.VMEM((KW, 16), f32)]
            + [pltpu.VMEM((KT,), jnp.int32), pltpu.VMEM((KT,), jnp.int32),
               pltpu.VMEM((KT, TW), f32),
               pltpu.VMEM((KT, 16), f32),
               pltpu.VMEM((16,), f32)]
            + 10 * [pltpu.SemaphoreType.DMA]
        ),
    )


def _gat_layer(prep, sc, combine, x_p, W, A_s, A_d, bias, src, dst):
    t, a_d, m2, init = prep(x_p, W, A_s, A_d)
    acc = sc(t, a_d, m2.reshape(16), src, dst, init)
    return combine(acc, bias)


_prep1 = _make_prep(8, 128, 128)
_prep2 = _make_prep(1, 64, 128)
_sc1 = _make_sc(8, 128, 96)
_sc2 = _make_sc(1, 64, 128)
_comb1 = _make_combine(8, 128, True)
_comb2 = _make_combine(1, 64, False)


def kernel(x, edge_index, W1, att_src1, att_dst1, bias1,
           W2, att_src2, att_dst2, bias2):
    src = edge_index[0].astype(jnp.int32)
    dst = edge_index[1].astype(jnp.int32)
    x_p = jnp.zeros((NP, D_IN), f32).at[:N].set(x)
    out1 = _gat_layer(_prep1, _sc1, _comb1, x_p, W1,
                      _att_mat(att_src1), _att_mat(att_dst1),
                      bias1.reshape(1, 128), src, dst)
    out2 = _gat_layer(_prep2, _sc2, _comb2, out1, W2,
                      _att_mat(att_src2), _att_mat(att_dst2),
                      bias2.reshape(1, 64), src, dst)
    return out2[:N]
